# baseline (speedup 1.0000x reference)
import functools

import jax
import jax.numpy as jnp
from jax import lax
from jax.experimental import pallas as pl
from jax.experimental.pallas import tpu as pltpu
from jax.experimental.pallas import tpu_sc as plsc

NC = 2
NS = 16
NW = NC * NS
LANES = 16

N = 10000
E = 320000
D = 128
H = 16
C = 2
G = 64

N_PAD = 10240
ROWS_PER_TILE = N_PAD // NS
EPR = ((E // NW) + 1023) // 1024 * 8
EPW = EPR * 128
E_PAD = EPW * NW
EPT = E_PAD // NS // 128
CH = 8
NCH = EPR // CH
NCH2 = NCH // 2

_MESH = plsc.VectorSubcoreMesh(
    core_axis_name="c", subcore_axis_name="s", num_cores=NC, num_subcores=NS
)


@functools.partial(
    pl.kernel,
    out_type=(
        jax.ShapeDtypeStruct((NC * N_PAD, H), jnp.float32),
        jax.ShapeDtypeStruct((N_PAD,), jnp.float32),
    ),
    mesh=_MESH,
    compiler_params=pltpu.CompilerParams(use_tc_tiling_on_sc=False,
                                         needs_layout_passes=False),
    scratch_types=[
        pltpu.VMEM((EPT, 128), jnp.int32),
        pltpu.VMEM((EPR, 128), jnp.int32),
        pltpu.VMEM((ROWS_PER_TILE, H), jnp.float32),
        pltpu.VMEM((ROWS_PER_TILE,), jnp.float32),
        pltpu.VMEM((ROWS_PER_TILE,), jnp.float32),
        pltpu.VMEM((2, CH * 128, H), jnp.float32),
        pltpu.VMEM((128,), jnp.float32),
        pltpu.VMEM_SHARED((N_PAD,), jnp.float32),
        pltpu.VMEM_SHARED((N_PAD, H), jnp.float32),
        pltpu.VMEM_SHARED((N_PAD, H), jnp.float32),
        pltpu.SemaphoreType.DMA,
        pltpu.SemaphoreType.DMA,
        pltpu.SemaphoreType.DMA,
        pltpu.SemaphoreType.DMA,
        pltpu.SemaphoreType.DMA,
    ],
)
def _sc_kernel(src_hbm, dst_hbm, h_hbm, ones_hbm, zflat_hbm, zrows_hbm,
               s_out, dinv_out,
               dstv, srcv, hv, degv, dinvv, rows_v, ones_v,
               deg_sh, hp_sh, s_sh,
               dsem, gsem_a, gsem_b, ssem_a, ssem_b):
    c = lax.axis_index("c")
    s = lax.axis_index("s")
    wid = s * NC + c
    base = s * ROWS_PER_TILE

    pltpu.sync_copy(zflat_hbm, deg_sh.at[pl.ds(base, ROWS_PER_TILE)])
    pltpu.sync_copy(zrows_hbm, s_sh.at[pl.ds(base, ROWS_PER_TILE)])
    pltpu.sync_copy(ones_hbm, ones_v)
    pltpu.sync_copy(dst_hbm.at[pl.ds(s * EPT, EPT)], dstv)
    pltpu.sync_copy(src_hbm.at[pl.ds(wid * EPR, EPR)], srcv)
    pltpu.sync_copy(h_hbm.at[pl.ds(base, ROWS_PER_TILE)], hv)
    plsc.subcore_barrier()

    def deg_body(j, carry):
        for r in range(8):
            pltpu.async_copy(ones_v, deg_sh.at[dstv.at[8 * j + r]], dsem,
                             add=True)
        for r in range(8):
            pltpu.make_async_copy(
                zflat_hbm.at[pl.ds(0, 128)], ones_v, dsem
            ).wait()
        return carry

    lax.fori_loop(0, EPT // 8, deg_body, 0)
    plsc.subcore_barrier()

    pltpu.sync_copy(deg_sh.at[pl.ds(base, ROWS_PER_TILE)], degv)
    for g in range(ROWS_PER_TILE // LANES):
        x = degv[pl.ds(g * LANES, LANES)] + 1.0
        y = plsc.bitcast(
            jnp.int32(0x5F3759DF)
            - lax.shift_right_logical(plsc.bitcast(x, jnp.int32), 1),
            jnp.float32)
        for _ in range(3):
            y = y * (1.5 - 0.5 * x * y * y)
        dinvv[pl.ds(g * LANES, LANES)] = y

    def scale_body(j, carry):
        for r in range(8):
            row = 8 * j + r
            sc = plsc.load_gather(
                dinvv, [jnp.full((LANES,), row, jnp.int32)])
            hv[row] = hv[row] * sc
        return carry

    lax.fori_loop(0, ROWS_PER_TILE // 8, scale_body, 0)
    pltpu.sync_copy(hv, hp_sh.at[pl.ds(base, ROWS_PER_TILE)])

    @pl.when(c == 0)
    def _():
        pltpu.sync_copy(dinvv, dinv_out.at[pl.ds(base, ROWS_PER_TILE)])

    plsc.subcore_barrier()

    gsems = (gsem_a, gsem_b)
    ssems = (ssem_a, ssem_b)

    def fire_g(chunk, buf):
        for r in range(CH):
            pltpu.async_copy(
                hp_sh.at[srcv.at[chunk * CH + r]],
                rows_v.at[buf, pl.ds(r * 128, 128)],
                gsems[buf],
            )

    def drain(sem, buf):
        for r in range(CH):
            pltpu.make_async_copy(
                h_hbm.at[pl.ds(0, 128)],
                rows_v.at[buf, pl.ds(r * 128, 128)],
                sem,
            ).wait()

    def fire_s(chunk, buf):
        for r in range(CH):
            pltpu.async_copy(
                rows_v.at[buf, pl.ds(r * 128, 128)],
                s_sh.at[dstv.at[c * EPR + chunk * CH + r]],
                ssems[buf],
                add=True,
            )

    fire_g(0, 0)

    def body(j2, carry):
        ca = 2 * j2
        cb = 2 * j2 + 1
        drain(gsems[0], 0)

        @pl.when(j2 > 0)
        def _():
            drain(ssems[1], 1)

        fire_g(cb, 1)
        fire_s(ca, 0)
        drain(gsems[1], 1)
        drain(ssems[0], 0)

        @pl.when(j2 < NCH2 - 1)
        def _():
            fire_g(ca + 2, 0)

        fire_s(cb, 1)
        return carry

    lax.fori_loop(0, NCH2, body, 0)
    drain(ssems[1], 1)
    plsc.subcore_barrier()
    pltpu.sync_copy(
        s_sh.at[pl.ds(base, ROWS_PER_TILE)],
        s_out.at[pl.ds(c * N_PAD + base, ROWS_PER_TILE)],
    )


def _prep_body(x_ref, w_ref, h_ref):
    h = jnp.dot(x_ref[...], w_ref[...], preferred_element_type=jnp.float32)
    h_ref[...] = jnp.concatenate(
        [h, jnp.zeros((N_PAD - N, H), jnp.float32)], axis=0)


def _prep_call(x, w):
    return pl.pallas_call(
        _prep_body,
        out_shape=jax.ShapeDtypeStruct((N_PAD, H), jnp.float32),
    )(x, w)


def _final_body(sp_ref, h_ref, dinv_ref, batch_ref, b_ref, wc_ref, bc_ref,
                out_ref):
    ssum = (lax.slice(sp_ref[0], (0, 0), (N, H))
            + lax.slice(sp_ref[1], (0, 0), (N, H)))
    h_n = lax.slice(h_ref[...], (0, 0), (N, H))
    dinv = lax.slice(dinv_ref[...], (0, 0), (N, 1))
    a = dinv * ssum + (dinv * dinv) * h_n + b_ref[...]
    hr = jnp.maximum(a, 0.0)
    iota = lax.broadcasted_iota(jnp.int32, (N, G), 1)
    onehot = (batch_ref[...] == iota).astype(jnp.float32)
    ps = lax.dot_general(onehot, hr, (((0,), (0,)), ((), ())),
                         preferred_element_type=jnp.float32)
    pc = lax.dot_general(onehot, jnp.ones((N, 1), jnp.float32),
                         (((0,), (0,)), ((), ())),
                         preferred_element_type=jnp.float32)
    pooled = ps / jnp.maximum(pc, 1.0)
    out_ref[...] = (
        jnp.dot(pooled, wc_ref[...], preferred_element_type=jnp.float32)
        + bc_ref[...]
    )


def _final_call(s_p, h_pad, dinv2, batch2, b, wc, bc):
    return pl.pallas_call(
        _final_body,
        out_shape=jax.ShapeDtypeStruct((G, C), jnp.float32),
    )(s_p, h_pad, dinv2, batch2, b.reshape(1, H), wc, bc.reshape(1, C))


def kernel(x, edge_index, batch, W, b, Wc, bc):
    src = edge_index[0]
    dst = edge_index[1]
    pad_e = E_PAD - E
    src_p = jnp.concatenate(
        [src, jnp.zeros((pad_e,), jnp.int32)]).reshape(E_PAD // 128, 128)
    dst_p = jnp.concatenate(
        [dst, jnp.full((pad_e,), N, jnp.int32)]).reshape(E_PAD // 128, 128)
    batch2 = batch.reshape(N, 1)

    ones128 = jnp.ones((128,), jnp.float32)
    zeros_flat = jnp.zeros((ROWS_PER_TILE,), jnp.float32)
    zeros_rows = jnp.zeros((ROWS_PER_TILE, H), jnp.float32)

    h_pad = _prep_call(x, W)
    s_p, dinv = _sc_kernel(src_p, dst_p, h_pad, ones128, zeros_flat,
                           zeros_rows)
    return _final_call(s_p.reshape(NC, N_PAD, H), h_pad,
                       dinv.reshape(N_PAD, 1), batch2, b, Wc, bc)


# speedup vs baseline: 1.0026x; 1.0026x over previous
import functools

import jax
import jax.numpy as jnp
from jax import lax
from jax.experimental import pallas as pl
from jax.experimental.pallas import tpu as pltpu
from jax.experimental.pallas import tpu_sc as plsc

NC = 2
NS = 16
NW = NC * NS
LANES = 16

N = 10000
E = 320000
D = 128
H = 16
C = 2
G = 64

N_PAD = 10240
ROWS_PER_TILE = N_PAD // NS
EPR = ((E // NW) + 1023) // 1024 * 8
EPW = EPR * 128
E_PAD = EPW * NW
EPT = E_PAD // NS // 128
CH = 4
NCH = EPR // CH
NCH2 = NCH // 2

_MESH = plsc.VectorSubcoreMesh(
    core_axis_name="c", subcore_axis_name="s", num_cores=NC, num_subcores=NS
)


@functools.partial(
    pl.kernel,
    out_type=(
        jax.ShapeDtypeStruct((NC * N_PAD, H), jnp.float32),
        jax.ShapeDtypeStruct((N_PAD,), jnp.float32),
    ),
    mesh=_MESH,
    compiler_params=pltpu.CompilerParams(use_tc_tiling_on_sc=False,
                                         needs_layout_passes=False),
    scratch_types=[
        pltpu.VMEM((EPT, 128), jnp.int32),
        pltpu.VMEM((EPR, 128), jnp.int32),
        pltpu.VMEM((ROWS_PER_TILE, H), jnp.float32),
        pltpu.VMEM((ROWS_PER_TILE,), jnp.float32),
        pltpu.VMEM((ROWS_PER_TILE,), jnp.float32),
        pltpu.VMEM((2, CH * 128, H), jnp.float32),
        pltpu.VMEM((128,), jnp.float32),
        pltpu.VMEM_SHARED((N_PAD,), jnp.float32),
        pltpu.VMEM_SHARED((N_PAD, H), jnp.float32),
        pltpu.VMEM_SHARED((N_PAD, H), jnp.float32),
        pltpu.SemaphoreType.DMA,
        pltpu.SemaphoreType.DMA,
        pltpu.SemaphoreType.DMA,
        pltpu.SemaphoreType.DMA,
        pltpu.SemaphoreType.DMA,
    ],
)
def _sc_kernel(src_hbm, dst_hbm, h_hbm, ones_hbm, zflat_hbm, zrows_hbm,
               s_out, dinv_out,
               dstv, srcv, hv, degv, dinvv, rows_v, ones_v,
               deg_sh, hp_sh, s_sh,
               dsem, gsem_a, gsem_b, ssem_a, ssem_b):
    c = lax.axis_index("c")
    s = lax.axis_index("s")
    wid = s * NC + c
    base = s * ROWS_PER_TILE

    pltpu.sync_copy(zflat_hbm, deg_sh.at[pl.ds(base, ROWS_PER_TILE)])
    pltpu.sync_copy(zrows_hbm, s_sh.at[pl.ds(base, ROWS_PER_TILE)])
    pltpu.sync_copy(ones_hbm, ones_v)
    pltpu.sync_copy(dst_hbm.at[pl.ds(s * EPT, EPT)], dstv)
    pltpu.sync_copy(src_hbm.at[pl.ds(wid * EPR, EPR)], srcv)
    pltpu.sync_copy(h_hbm.at[pl.ds(base, ROWS_PER_TILE)], hv)
    plsc.subcore_barrier()

    def deg_body(j, carry):
        for r in range(8):
            pltpu.async_copy(ones_v, deg_sh.at[dstv.at[8 * j + r]], dsem,
                             add=True)
        for r in range(8):
            pltpu.make_async_copy(
                zflat_hbm.at[pl.ds(0, 128)], ones_v, dsem
            ).wait()
        return carry

    lax.fori_loop(0, EPT // 8, deg_body, 0)
    plsc.subcore_barrier()

    pltpu.sync_copy(deg_sh.at[pl.ds(base, ROWS_PER_TILE)], degv)
    for g in range(ROWS_PER_TILE // LANES):
        x = degv[pl.ds(g * LANES, LANES)] + 1.0
        y = plsc.bitcast(
            jnp.int32(0x5F3759DF)
            - lax.shift_right_logical(plsc.bitcast(x, jnp.int32), 1),
            jnp.float32)
        for _ in range(3):
            y = y * (1.5 - 0.5 * x * y * y)
        dinvv[pl.ds(g * LANES, LANES)] = y

    def scale_body(j, carry):
        for r in range(8):
            row = 8 * j + r
            sc = plsc.load_gather(
                dinvv, [jnp.full((LANES,), row, jnp.int32)])
            hv[row] = hv[row] * sc
        return carry

    lax.fori_loop(0, ROWS_PER_TILE // 8, scale_body, 0)
    pltpu.sync_copy(hv, hp_sh.at[pl.ds(base, ROWS_PER_TILE)])

    @pl.when(c == 0)
    def _():
        pltpu.sync_copy(dinvv, dinv_out.at[pl.ds(base, ROWS_PER_TILE)])

    plsc.subcore_barrier()

    gsems = (gsem_a, gsem_b)
    ssems = (ssem_a, ssem_b)

    def fire_g(chunk, buf):
        for r in range(CH):
            pltpu.async_copy(
                hp_sh.at[srcv.at[chunk * CH + r]],
                rows_v.at[buf, pl.ds(r * 128, 128)],
                gsems[buf],
            )

    def drain(sem, buf):
        for r in range(CH):
            pltpu.make_async_copy(
                h_hbm.at[pl.ds(0, 128)],
                rows_v.at[buf, pl.ds(r * 128, 128)],
                sem,
            ).wait()

    def fire_s(chunk, buf):
        for r in range(CH):
            pltpu.async_copy(
                rows_v.at[buf, pl.ds(r * 128, 128)],
                s_sh.at[dstv.at[c * EPR + chunk * CH + r]],
                ssems[buf],
                add=True,
            )

    fire_g(0, 0)

    def body(j2, carry):
        ca = 2 * j2
        cb = 2 * j2 + 1
        drain(gsems[0], 0)

        @pl.when(j2 > 0)
        def _():
            drain(ssems[1], 1)

        fire_g(cb, 1)
        fire_s(ca, 0)
        drain(gsems[1], 1)
        drain(ssems[0], 0)

        @pl.when(j2 < NCH2 - 1)
        def _():
            fire_g(ca + 2, 0)

        fire_s(cb, 1)
        return carry

    lax.fori_loop(0, NCH2, body, 0)
    drain(ssems[1], 1)
    plsc.subcore_barrier()
    pltpu.sync_copy(
        s_sh.at[pl.ds(base, ROWS_PER_TILE)],
        s_out.at[pl.ds(c * N_PAD + base, ROWS_PER_TILE)],
    )


def _prep_body(x_ref, w_ref, h_ref):
    h = jnp.dot(x_ref[...], w_ref[...], preferred_element_type=jnp.float32)
    h_ref[...] = jnp.concatenate(
        [h, jnp.zeros((N_PAD - N, H), jnp.float32)], axis=0)


def _prep_call(x, w):
    return pl.pallas_call(
        _prep_body,
        out_shape=jax.ShapeDtypeStruct((N_PAD, H), jnp.float32),
    )(x, w)


def _final_body(sp_ref, h_ref, dinv_ref, batch_ref, b_ref, wc_ref, bc_ref,
                out_ref):
    ssum = (lax.slice(sp_ref[0], (0, 0), (N, H))
            + lax.slice(sp_ref[1], (0, 0), (N, H)))
    h_n = lax.slice(h_ref[...], (0, 0), (N, H))
    dinv = lax.slice(dinv_ref[...], (0, 0), (N, 1))
    a = dinv * ssum + (dinv * dinv) * h_n + b_ref[...]
    hr = jnp.maximum(a, 0.0)
    iota = lax.broadcasted_iota(jnp.int32, (N, G), 1)
    onehot = (batch_ref[...] == iota).astype(jnp.float32)
    ps = lax.dot_general(onehot, hr, (((0,), (0,)), ((), ())),
                         preferred_element_type=jnp.float32)
    pc = lax.dot_general(onehot, jnp.ones((N, 1), jnp.float32),
                         (((0,), (0,)), ((), ())),
                         preferred_element_type=jnp.float32)
    pooled = ps / jnp.maximum(pc, 1.0)
    out_ref[...] = (
        jnp.dot(pooled, wc_ref[...], preferred_element_type=jnp.float32)
        + bc_ref[...]
    )


def _final_call(s_p, h_pad, dinv2, batch2, b, wc, bc):
    return pl.pallas_call(
        _final_body,
        out_shape=jax.ShapeDtypeStruct((G, C), jnp.float32),
    )(s_p, h_pad, dinv2, batch2, b.reshape(1, H), wc, bc.reshape(1, C))


def kernel(x, edge_index, batch, W, b, Wc, bc):
    src = edge_index[0]
    dst = edge_index[1]
    pad_e = E_PAD - E
    src_p = jnp.concatenate(
        [src, jnp.zeros((pad_e,), jnp.int32)]).reshape(E_PAD // 128, 128)
    dst_p = jnp.concatenate(
        [dst, jnp.full((pad_e,), N, jnp.int32)]).reshape(E_PAD // 128, 128)
    batch2 = batch.reshape(N, 1)

    ones128 = jnp.ones((128,), jnp.float32)
    zeros_flat = jnp.zeros((ROWS_PER_TILE,), jnp.float32)
    zeros_rows = jnp.zeros((ROWS_PER_TILE, H), jnp.float32)

    h_pad = _prep_call(x, W)
    s_p, dinv = _sc_kernel(src_p, dst_p, h_pad, ones128, zeros_flat,
                           zeros_rows)
    return _final_call(s_p.reshape(NC, N_PAD, H), h_pad,
                       dinv.reshape(N_PAD, 1), batch2, b, Wc, bc)
